# baseline (speedup 1.0000x reference)
import functools

import jax
import jax.numpy as jnp
from jax import lax
from jax.experimental import pallas as pl
from jax.experimental.pallas import tpu as pltpu
from jax.experimental.pallas import tpu_sc as plsc


def _softmax_t_body(x_ref, o_ref):
    x = x_ref[...]
    m = jnp.max(x, axis=1, keepdims=True)
    e = jnp.exp(x - m)
    o_ref[...] = (e / jnp.sum(e, axis=1, keepdims=True)).T


def _softmax_t_tc(x):
    T, C = x.shape
    return pl.pallas_call(
        _softmax_t_body,
        out_shape=jax.ShapeDtypeStruct((C, T), x.dtype),
    )(x)


def _make_gather_sc(B, S, C, T):
    CT = C // 8
    BT = B // 128
    BG = B // 16
    n_workers = 32
    k_max = (CT + n_workers - 1) // n_workers
    mesh = plsc.VectorSubcoreMesh(core_axis_name="c", subcore_axis_name="s")
    nc = 2

    @functools.partial(
        pl.kernel,
        mesh=mesh,
        compiler_params=pltpu.CompilerParams(
            use_tc_tiling_on_sc=False, needs_layout_passes=False
        ),
        out_type=jax.ShapeDtypeStruct((S, CT, BT, 8, 128), jnp.float32),
        scratch_types=[
            pltpu.VMEM((S, BG, 16), jnp.int32),
            pltpu.VMEM((k_max * 8, T), jnp.float32),
            pltpu.VMEM((BT, 8, 128), jnp.float32),
            pltpu.VMEM((BT, 8, 128), jnp.float32),
            pltpu.SemaphoreType.DMA,
            pltpu.SemaphoreType.DMA,
        ],
    )
    def gather_kernel(probs_t, idx_t, out_hbm, idx_v, tab_v, buf0, buf1,
                      sem0, sem1):
        wid = lax.axis_index("s") * nc + lax.axis_index("c")
        bufs = (buf0, buf1)
        sems = (sem0, sem1)
        pltpu.sync_copy(idx_t, idx_v)
        for k in range(k_max):
            ct = wid + n_workers * k

            @pl.when(ct < CT)
            def _():
                pltpu.sync_copy(
                    probs_t.at[pl.ds(ct * 8, 8)], tab_v.at[pl.ds(k * 8, 8)]
                )

        def unit(k, ct, s, b, first):
            if not first:
                pltpu.make_async_copy(bufs[b], out_hbm.at[0, 0], sems[b]).wait()
            rowbase = k * 8
            rows = [jnp.full((16,), rowbase + cq, jnp.int32) for cq in range(8)]

            @pl.loop(0, BT, unroll=4)
            def _(bt):
                idxs = [idx_v[s, bt * 8 + g] for g in range(8)]
                prev = None
                for g in range(8):
                    cur = [
                        plsc.load_gather(tab_v, [rows[cq], idxs[g]])
                        for cq in range(8)
                    ]
                    if prev is not None:
                        for cq in range(8):
                            bufs[b][bt, cq, pl.ds((g - 1) * 16, 16)] = prev[cq]
                    prev = cur
                for cq in range(8):
                    bufs[b][bt, cq, pl.ds(7 * 16, 16)] = prev[cq]

            pltpu.async_copy(bufs[b], out_hbm.at[s, ct], sems[b])

        half_s = S // 2
        unit(0, wid, 0, 0, True)
        unit(0, wid, 1, 1, True)

        @pl.loop(1, k_max * half_s)
        def _(u):
            k = u // half_s
            sp = (u % half_s) * 2
            ct = wid + n_workers * k

            @pl.when(ct < CT)
            def _():
                unit(k, ct, sp, 0, False)
                unit(k, ct, sp + 1, 1, False)

        for b in range(2):
            pltpu.make_async_copy(bufs[b], out_hbm.at[0, 0], sems[b]).wait()

    return gather_kernel


def kernel(inputs_hour, catid_time_matrix):
    B, S = inputs_hour.shape
    T, C = catid_time_matrix.shape
    CT = C // 8
    BT = B // 128

    probs_t = _softmax_t_tc(catid_time_matrix)
    idx_t = (
        inputs_hour.astype(jnp.int32).T.reshape(S, B // 16, 16)
    )
    gather = _make_gather_sc(B, S, C, T)
    ot5 = gather(probs_t, idx_t)
    out = jnp.transpose(ot5, (2, 4, 0, 1, 3))
    out = out.reshape(B, S, CT, 8)
    return out.reshape(B, S, C)


# speedup vs baseline: 1.0093x; 1.0093x over previous
import functools

import jax
import jax.numpy as jnp
from jax import lax
from jax.experimental import pallas as pl
from jax.experimental.pallas import tpu as pltpu
from jax.experimental.pallas import tpu_sc as plsc


def _softmax_t_body(x_ref, o_ref):
    x = x_ref[...]
    m = jnp.max(x, axis=1, keepdims=True)
    e = jnp.exp(x - m)
    o_ref[...] = (e / jnp.sum(e, axis=1, keepdims=True)).T


def _softmax_t_tc(x):
    T, C = x.shape
    return pl.pallas_call(
        _softmax_t_body,
        out_shape=jax.ShapeDtypeStruct((C, T), x.dtype),
    )(x)


def _make_gather_sc(B, S, C, T):
    CT = C // 8
    BT = B // 128
    BG = B // 16
    n_workers = 32
    k_max = (CT + n_workers - 1) // n_workers
    mesh = plsc.VectorSubcoreMesh(core_axis_name="c", subcore_axis_name="s")
    nc = 2

    @functools.partial(
        pl.kernel,
        mesh=mesh,
        compiler_params=pltpu.CompilerParams(
            use_tc_tiling_on_sc=False, needs_layout_passes=False
        ),
        out_type=jax.ShapeDtypeStruct((S, CT, BT, 8, 128), jnp.float32),
        scratch_types=[
            pltpu.VMEM((S, BG, 16), jnp.int32),
            pltpu.VMEM((k_max * 8, T), jnp.float32),
            pltpu.VMEM((BT, 8, 128), jnp.float32),
            pltpu.VMEM((BT, 8, 128), jnp.float32),
            pltpu.SemaphoreType.DMA,
            pltpu.SemaphoreType.DMA,
        ],
    )
    def gather_kernel(probs_t, idx_t, out_hbm, idx_v, tab_v, buf0, buf1,
                      sem0, sem1):
        wid = lax.axis_index("s") * nc + lax.axis_index("c")
        bufs = (buf0, buf1)
        sems = (sem0, sem1)
        pltpu.sync_copy(idx_t, idx_v)
        for k in range(k_max):
            ct = wid + n_workers * k

            @pl.when(ct < CT)
            def _():
                pltpu.sync_copy(
                    probs_t.at[pl.ds(ct * 8, 8)], tab_v.at[pl.ds(k * 8, 8)]
                )

        def unit(k, ct, s, b, first):
            if not first:
                pltpu.make_async_copy(bufs[b], out_hbm.at[0, 0], sems[b]).wait()
            rowbase = k * 8
            rows = [jnp.full((16,), rowbase + cq, jnp.int32) for cq in range(8)]

            @pl.loop(0, BT, unroll=2)
            def _(bt):
                idxs = [idx_v[s, bt * 8 + g] for g in range(8)]
                prev = None
                for g in range(8):
                    cur = [
                        plsc.load_gather(tab_v, [rows[cq], idxs[g]])
                        for cq in range(8)
                    ]
                    if prev is not None:
                        for cq in range(8):
                            bufs[b][bt, cq, pl.ds((g - 1) * 16, 16)] = prev[cq]
                    prev = cur
                for cq in range(8):
                    bufs[b][bt, cq, pl.ds(7 * 16, 16)] = prev[cq]

            pltpu.async_copy(bufs[b], out_hbm.at[s, ct], sems[b])

        half_s = S // 2
        unit(0, wid, 0, 0, True)
        unit(0, wid, 1, 1, True)

        @pl.loop(1, k_max * half_s)
        def _(u):
            k = u // half_s
            sp = (u % half_s) * 2
            ct = wid + n_workers * k

            @pl.when(ct < CT)
            def _():
                unit(k, ct, sp, 0, False)
                unit(k, ct, sp + 1, 1, False)

        for b in range(2):
            pltpu.make_async_copy(bufs[b], out_hbm.at[0, 0], sems[b]).wait()

    return gather_kernel


def kernel(inputs_hour, catid_time_matrix):
    B, S = inputs_hour.shape
    T, C = catid_time_matrix.shape
    CT = C // 8
    BT = B // 128

    probs_t = _softmax_t_tc(catid_time_matrix)
    idx_t = (
        inputs_hour.astype(jnp.int32).T.reshape(S, B // 16, 16)
    )
    gather = _make_gather_sc(B, S, C, T)
    ot5 = gather(probs_t, idx_t)
    out = jnp.transpose(ot5, (2, 4, 0, 1, 3))
    out = out.reshape(B, S, CT, 8)
    return out.reshape(B, S, C)
